# baseline (speedup 1.0000x reference)
import functools

import jax
import jax.numpy as jnp
from jax.experimental import pallas as pl
from jax.experimental.pallas import tpu as pltpu

_C = 21
_THRESH = 0.5
_V0, _V1 = 0.1, 0.2
_S = 1.0 / (_V0 * _V1)
_L = 128
_SPB = 2
_PM = 8704


def _sl1(d):
    a = jnp.abs(d)
    return jnp.where(a < 1.0, 0.5 * a * a, a - 0.5)


def _bitsel(vals, key):
    nodes = list(vals)
    d = 0
    while len(nodes) > 1:
        bit = (key & (1 << d)) != 0
        nxt = [jnp.where(bit, nodes[2 * i + 1], nodes[2 * i])
               for i in range(len(nodes) // 2)]
        if len(nodes) % 2:
            nxt.append(nodes[-1])
        nodes = nxt
        d += 1
    return nodes[0]


def _tree_add(terms):
    while len(terms) > 1:
        nxt = [terms[i] + terms[i + 1] for i in range(0, len(terms) - 1, 2)]
        if len(terms) % 2:
            nxt.append(terms[-1])
        terms = nxt
    return terms[0]


def _first_max_fold(ovs):
    nodes = [(ovs[j], j) for j in range(len(ovs))]
    while len(nodes) > 1:
        nxt = []
        for i in range(0, len(nodes) - 1, 2):
            av, ai = nodes[i]
            bv, bi = nodes[i + 1]
            keep = av >= bv
            nxt.append((jnp.where(keep, av, bv), jnp.where(keep, ai, bi)))
        if len(nodes) % 2:
            nxt.append(nodes[-1])
        nodes = nxt
    return nodes[0]


def _last_true_fold(pairs):
    nodes = list(pairs)
    while len(nodes) > 1:
        nxt = []
        for i in range(0, len(nodes) - 1, 2):
            ah, ai = nodes[i]
            bh, bi = nodes[i + 1]
            nxt.append((ah | bh, jnp.where(bh, bi, ai)))
        if len(nodes) % 2:
            nxt.append(nodes[-1])
        nodes = nxt
    return nodes[0]


def _part_losses(truth, ovs, bps, pri, idx2, valid, loc, fc, cf, nobj):
    pcx, pcy, pw, ph = pri[0], pri[1], pri[2], pri[3]

    bto, bti = _first_max_fold(ovs)
    ohas, oidx = _last_true_fold([(idx2 == bps[j], j) for j in range(nobj)])
    bto = jnp.where(ohas, 2.0, bto)
    bti = jnp.where(ohas, oidx, bti)

    tv = truth
    mc = [_bitsel(tv[k], bti) for k in range(12)]
    cls = _bitsel([tv[12][j] + 1.0 for j in range(nobj)], bti)
    conf_t = jnp.where(bto < _THRESH, 0, cls.astype(jnp.int32))
    pos = conf_t > 0
    if valid is not None:
        pos = pos & valid

    mx1, my1, mx2, my2 = mc[0], mc[1], mc[2], mc[3]
    gcx = ((mx1 + mx2) / 2.0 - pcx) / (_V0 * pw)
    gcy = ((my1 + my2) / 2.0 - pcy) / (_V0 * ph)
    gw = jnp.log((mx2 - mx1) / pw) / _V1
    gh = jnp.log((my2 - my1) / ph) / _V1
    ll_terms = ((_sl1(loc[0] - gcx) + _sl1(loc[1] - gcy))
                + (_sl1(loc[2] - gw) + _sl1(loc[3] - gh)))
    ll = jnp.sum(jnp.where(pos, ll_terms, 0.0))

    pxy = [pcx, pcy] * 4
    pwh = [pw, ph] * 4
    fc_terms = _tree_add(
        [_sl1(fc[k] - (mc[4 + k] - pxy[k]) / (_V0 * pwh[k]))
         for k in range(8)])
    lfc = jnp.sum(jnp.where(pos, fc_terms, 0.0))

    dw = pw * jnp.exp(loc[2] * _V1)
    dh = ph * jnp.exp(loc[3] * _V1)
    dx1 = pcx + loc[0] * _V0 * pw - dw / 2.0
    dy1 = pcy + loc[1] * _V0 * ph - dh / 2.0
    dx2 = dx1 + dw
    dy2 = dy1 + dh
    df = [pxy[k] + fc[k] * _V0 * pwh[k] for k in range(8)]
    b_terms = ((_sl1(jnp.tanh(dx1 - jnp.minimum(df[0], df[6])) * _S)
                + _sl1(jnp.tanh(dy1 - jnp.minimum(df[1], df[3])) * _S))
               + (_sl1(jnp.tanh(dx2 - jnp.maximum(df[2], df[4])) * _S)
                  + _sl1(jnp.tanh(dy2 - jnp.maximum(df[5], df[7])) * _S)))
    lb = jnp.sum(jnp.where(pos, b_terms, 0.0))

    mrow = cf[0]
    for c in range(1, _C):
        mrow = jnp.maximum(mrow, cf[c])
    lse = jnp.log(_tree_add([jnp.exp(cf[c] - mrow) for c in range(_C)])) + mrow
    ce = lse - _bitsel(cf, conf_t)
    ce_pos = jnp.sum(jnp.where(pos, ce, 0.0))
    npos = jnp.sum(pos.astype(jnp.int32)).astype(jnp.float32)

    npmask = pos if valid is None else (pos | jnp.logical_not(valid))
    lcm = jnp.maximum(jnp.where(npmask, 0.0, ce), 0.0)
    return ll, ce_pos, lfc, lb, npos, lcm


def _iou(ax1, ay1, ax2, ay2, aarea, pri, parea):
    px1, py1, px2, py2 = pri[4], pri[5], pri[6], pri[7]
    iw = jnp.maximum(jnp.minimum(ax2, px2) - jnp.maximum(ax1, px1), 0.0)
    ih = jnp.maximum(jnp.minimum(ay2, py2) - jnp.maximum(ay1, py1), 0.0)
    inter = iw * ih
    return inter / (aarea + parea - inter)


def _sample_body(tgt_ref, loc_ref, conf_ref, fc_ref, prim_ref, prit_ref,
                 row_ref, lcmm_ref, lcmt_ref, *, num_priors, nobj):
    RM = _PM // _L
    ntail = num_priors - _PM
    prim = [prim_ref[k] for k in range(8)]
    prit = [prit_ref[k] for k in range(8)]
    pam = (prim[6] - prim[4]) * (prim[7] - prim[5])
    pat = (prit[6] - prit[4]) * (prit[7] - prit[5])

    row_i = jax.lax.broadcasted_iota(jnp.int32, (RM, _L), 0)
    col_i = jax.lax.broadcasted_iota(jnp.int32, (RM, _L), 1)
    idx2m = row_i * _L + col_i
    colt = jax.lax.broadcasted_iota(jnp.int32, (1, _L), 1)
    idx2t = _PM + colt
    validt = colt < ntail

    r8 = jax.lax.broadcasted_iota(jnp.int32, (8, 128), 0)
    c8 = jax.lax.broadcasted_iota(jnp.int32, (8, 128), 1)
    z = jnp.zeros((8, 128), jnp.float32)
    first = c8 == 0

    for s in range(_SPB):
        locm = jnp.transpose(
            loc_ref[s, :_PM, :].reshape(RM, _L, 4), (2, 0, 1))
        cfm = jnp.transpose(
            conf_ref[s, :_PM, :].reshape(RM, _L, _C), (2, 0, 1))
        fcm = jnp.transpose(
            fc_ref[s, :_PM, :].reshape(RM, _L, 8), (2, 0, 1))
        loct = jnp.pad(jnp.transpose(loc_ref[s, _PM:num_priors, :], (1, 0)),
                       ((0, 0), (0, _L - ntail))).reshape(4, 1, _L)
        cft = jnp.pad(jnp.transpose(conf_ref[s, _PM:num_priors, :], (1, 0)),
                      ((0, 0), (0, _L - ntail))).reshape(_C, 1, _L)
        fct = jnp.pad(jnp.transpose(fc_ref[s, _PM:num_priors, :], (1, 0)),
                      ((0, 0), (0, _L - ntail))).reshape(8, 1, _L)

        truth = [[tgt_ref[s, j, k] for j in range(nobj)] for k in range(13)]
        ovs_m, ovs_t, bps = [], [], []
        for j in range(nobj):
            ax1, ay1 = truth[0][j], truth[1][j]
            ax2, ay2 = truth[2][j], truth[3][j]
            aarea = (ax2 - ax1) * (ay2 - ay1)
            ovm = _iou(ax1, ay1, ax2, ay2, aarea, prim, pam)
            ovt = _iou(ax1, ay1, ax2, ay2, aarea, prit, pat)
            ovt = jnp.where(validt, ovt, -1.0)
            m = jnp.maximum(jnp.max(ovm), jnp.max(ovt))
            bpm = jnp.min(jnp.where(ovm == m, idx2m, num_priors))
            bpt = jnp.min(jnp.where(ovt == m, idx2t, num_priors))
            bps.append(jnp.minimum(bpm, bpt))
            ovs_m.append(ovm)
            ovs_t.append(ovt)

        rm = _part_losses(truth, ovs_m, bps, prim, idx2m, None,
                          [locm[k] for k in range(4)],
                          [fcm[k] for k in range(8)],
                          [cfm[c] for c in range(_C)], nobj)
        rt = _part_losses(truth, ovs_t, bps, prit, idx2t, validt,
                          [loct[k, 0:1] for k in range(4)],
                          [fct[k, 0:1] for k in range(8)],
                          [cft[c, 0:1] for c in range(_C)], nobj)

        ll, ce_pos, lfc, lb, npos = (rm[i] + rt[i] for i in range(5))
        lcmm_ref[s] = rm[5]
        lcmt_ref[s] = rt[5]
        row_ref[s] = (jnp.where((r8 == 0) & first, ll, z)
                      + jnp.where((r8 == 1) & first, ce_pos, z)
                      + jnp.where((r8 == 2) & first, lfc, z)
                      + jnp.where((r8 == 3) & first, lb, z)
                      + jnp.where((r8 == 4) & first, npos, z))


def _topk_body(row_ref, lcmm_ref, lcmt_ref, out_ref, *, num_priors):
    sb = row_ref[...]
    lcm_m = lcmm_ref[...]
    lcm_t = lcmt_ref[...]
    B = sb.shape[0]
    npos = sb[:, 4, 0].astype(jnp.int32)
    kneg = jnp.minimum(3 * npos, num_priors - 1).reshape(B, 1, 1)

    um = pltpu.bitcast(lcm_m, jnp.int32)
    ut = pltpu.bitcast(lcm_t, jnp.int32)

    def _rowsum(x):
        return jnp.sum(jnp.sum(x, axis=1, keepdims=True), axis=2,
                       keepdims=True)

    def _cnt(mid):
        return (_rowsum((um >= mid).astype(jnp.int32))
                + _rowsum((ut >= mid).astype(jnp.int32)))

    def _bs(_, lohi):
        lo, hi = lohi
        mid = lo + ((hi - lo + 1) >> 1)
        ok = _cnt(mid) >= kneg
        return (jnp.where(ok, mid, lo), jnp.where(ok, hi, mid - 1))

    init = (jnp.zeros((B, 1, 1), jnp.int32),
            jnp.full((B, 1, 1), 0x7F7FFFFF, jnp.int32))
    t, _ = jax.lax.fori_loop(0, 31, _bs, init)
    cnt_gt = _rowsum((um > t).astype(jnp.int32)) \
        + _rowsum((ut > t).astype(jnp.int32))
    sum_gt = _rowsum(jnp.where(um > t, lcm_m, 0.0)) \
        + _rowsum(jnp.where(ut > t, lcm_t, 0.0))

    def _rowmax(x):
        return jnp.max(jnp.max(x, axis=1, keepdims=True), axis=2,
                       keepdims=True)

    tval = jnp.maximum(_rowmax(jnp.where(um == t, lcm_m, 0.0)),
                       _rowmax(jnp.where(ut == t, lcm_t, 0.0)))
    topk = sum_gt + (kneg - cnt_gt).astype(jnp.float32) * tval

    ll = jnp.sum(sb[:, 0, 0])
    lc = jnp.sum(sb[:, 1, 0]) + jnp.sum(topk)
    lfc = jnp.sum(sb[:, 2, 0])
    lb = jnp.sum(sb[:, 3, 0])
    n = jnp.sum(sb[:, 4, 0])

    r8 = jax.lax.broadcasted_iota(jnp.int32, (8, 128), 0)
    c8 = jax.lax.broadcasted_iota(jnp.int32, (8, 128), 1)
    z = jnp.zeros((8, 128), jnp.float32)
    first = c8 == 0
    out_ref[...] = (jnp.where((r8 == 0) & first, ll, z)
                    + jnp.where((r8 == 1) & first, lc, z)
                    + jnp.where((r8 == 2) & first, lfc, z)
                    + jnp.where((r8 == 3) & first, lb, z)
                    + jnp.where((r8 == 4) & first, n, z))


def kernel(loc_data, conf_data, priors, four_corners_data, targets):
    B, P, C = conf_data.shape
    nobj = targets.shape[1]
    RM = _PM // _L
    ntail = P - _PM

    pf = jnp.concatenate(
        (priors[:, :2] - priors[:, 2:] / 2.0,
         priors[:, :2] + priors[:, 2:] / 2.0), axis=1)
    priall = jnp.concatenate([priors, pf], axis=1).T
    prim = priall[:, :_PM].reshape(8, RM, _L)
    prit = jnp.pad(priall[:, _PM:], ((0, 0), (0, _L - ntail))) \
        .reshape(8, 1, _L)

    rows, lcm_m, lcm_t = pl.pallas_call(
        functools.partial(_sample_body, num_priors=P, nobj=nobj),
        grid=(B // _SPB,),
        in_specs=[
            pl.BlockSpec((_SPB, nobj, 13), lambda b: (b, 0, 0),
                         memory_space=pltpu.SMEM),
            pl.BlockSpec((_SPB, P, 4), lambda b: (b, 0, 0)),
            pl.BlockSpec((_SPB, P, C), lambda b: (b, 0, 0)),
            pl.BlockSpec((_SPB, P, 8), lambda b: (b, 0, 0)),
            pl.BlockSpec((8, RM, _L), lambda b: (0, 0, 0)),
            pl.BlockSpec((8, 1, _L), lambda b: (0, 0, 0)),
        ],
        out_specs=[
            pl.BlockSpec((_SPB, 8, 128), lambda b: (b, 0, 0)),
            pl.BlockSpec((_SPB, RM, _L), lambda b: (b, 0, 0)),
            pl.BlockSpec((_SPB, 1, _L), lambda b: (b, 0, 0)),
        ],
        out_shape=[
            jax.ShapeDtypeStruct((B, 8, 128), jnp.float32),
            jax.ShapeDtypeStruct((B, RM, _L), jnp.float32),
            jax.ShapeDtypeStruct((B, 1, _L), jnp.float32),
        ],
        compiler_params=pltpu.CompilerParams(
            dimension_semantics=("parallel",)),
    )(targets, loc_data, conf_data, four_corners_data, prim, prit)

    out = pl.pallas_call(
        functools.partial(_topk_body, num_priors=P),
        out_shape=jax.ShapeDtypeStruct((8, 128), jnp.float32),
    )(rows, lcm_m, lcm_t)

    n = out[4, 0]
    return (out[0, 0] / n, out[1, 0] / n, out[2, 0] / n, out[3, 0] / n)


# speedup vs baseline: 2.0752x; 2.0752x over previous
import functools

import jax
import jax.numpy as jnp
from jax.experimental import pallas as pl
from jax.experimental.pallas import tpu as pltpu

_C = 21
_THRESH = 0.5
_V0, _V1 = 0.1, 0.2
_S = 1.0 / (_V0 * _V1)
_L = 128
_SPB = 2


def _sl1(d):
    a = jnp.abs(d)
    return jnp.where(a < 1.0, 0.5 * a * a, a - 0.5)


def _bitsel(vals, key):
    nodes = list(vals)
    d = 0
    while len(nodes) > 1:
        bit = (key & (1 << d)) != 0
        nxt = [jnp.where(bit, nodes[2 * i + 1], nodes[2 * i])
               for i in range(len(nodes) // 2)]
        if len(nodes) % 2:
            nxt.append(nodes[-1])
        nodes = nxt
        d += 1
    return nodes[0]


def _tree_add(terms):
    while len(terms) > 1:
        nxt = [terms[i] + terms[i + 1] for i in range(0, len(terms) - 1, 2)]
        if len(terms) % 2:
            nxt.append(terms[-1])
        terms = nxt
    return terms[0]


def _one_sample(s, tgt_ref, loc_ref, conf_ref, fc_ref, pri, idx2, valid,
                num_priors, nobj):
    pcx, pcy, pw, ph, px1, py1, px2, py2, parea = pri
    R, L = idx2.shape

    ovs = []
    bps = []
    for j in range(nobj):
        ax1 = tgt_ref[s, j, 0]
        ay1 = tgt_ref[s, j, 1]
        ax2 = tgt_ref[s, j, 2]
        ay2 = tgt_ref[s, j, 3]
        aarea = (ax2 - ax1) * (ay2 - ay1)
        iw = jnp.maximum(jnp.minimum(ax2, px2) - jnp.maximum(ax1, px1), 0.0)
        ih = jnp.maximum(jnp.minimum(ay2, py2) - jnp.maximum(ay1, py1), 0.0)
        inter = iw * ih
        ov = inter / (aarea + parea - inter)
        m = jnp.max(ov)
        bps.append(jnp.min(jnp.where(ov == m, idx2, num_priors)))
        ovs.append(ov)

    nodes = [(ovs[j], j) for j in range(nobj)]
    while len(nodes) > 1:
        nxt = []
        for i in range(0, len(nodes) - 1, 2):
            av, ai = nodes[i]
            bv, bi = nodes[i + 1]
            keep = av >= bv
            nxt.append((jnp.where(keep, av, bv), jnp.where(keep, ai, bi)))
        if len(nodes) % 2:
            nxt.append(nodes[-1])
        nodes = nxt
    bto, bti = nodes[0]

    onodes = [(idx2 == bps[j], j) for j in range(nobj)]
    while len(onodes) > 1:
        nxt = []
        for i in range(0, len(onodes) - 1, 2):
            ah, ai = onodes[i]
            bh, bi = onodes[i + 1]
            nxt.append((ah | bh, jnp.where(bh, bi, ai)))
        if len(onodes) % 2:
            nxt.append(onodes[-1])
        onodes = nxt
    ohas, oidx = onodes[0]
    bto = jnp.where(ohas, 2.0, bto)
    bti = jnp.where(ohas, oidx, bti)

    tv = [[tgt_ref[s, j, k] for j in range(nobj)] for k in range(13)]
    mc = [_bitsel(tv[k], bti) for k in range(12)]
    cls = _bitsel([tv[12][j] + 1.0 for j in range(nobj)], bti)
    conf_t = jnp.where(bto < _THRESH, 0, cls.astype(jnp.int32))
    pos = (conf_t > 0) & valid

    mx1, my1, mx2, my2 = mc[0], mc[1], mc[2], mc[3]
    loc = [loc_ref[s, k] for k in range(4)]
    ivw = 1.0 / (_V0 * pw)
    ivh = 1.0 / (_V0 * ph)
    gcx = ((mx1 + mx2) / 2.0 - pcx) * ivw
    gcy = ((my1 + my2) / 2.0 - pcy) * ivh
    gw = jnp.log((mx2 - mx1) / pw) / _V1
    gh = jnp.log((my2 - my1) / ph) / _V1
    ll_terms = ((_sl1(loc[0] - gcx) + _sl1(loc[1] - gcy))
                + (_sl1(loc[2] - gw) + _sl1(loc[3] - gh)))
    ll = jnp.sum(jnp.where(pos, ll_terms, 0.0))

    fc = [fc_ref[s, k] for k in range(8)]
    pxy = [pcx, pcy] * 4
    pwh = [pw, ph] * 4
    ivwh = [ivw, ivh] * 4
    fc_terms = _tree_add(
        [_sl1(fc[k] - (mc[4 + k] - pxy[k]) * ivwh[k])
         for k in range(8)])
    lfc = jnp.sum(jnp.where(pos, fc_terms, 0.0))

    dw = pw * jnp.exp(loc[2] * _V1)
    dh = ph * jnp.exp(loc[3] * _V1)
    dx1 = pcx + loc[0] * _V0 * pw - dw / 2.0
    dy1 = pcy + loc[1] * _V0 * ph - dh / 2.0
    dx2 = dx1 + dw
    dy2 = dy1 + dh
    df = [pxy[k] + fc[k] * _V0 * pwh[k] for k in range(8)]
    b_terms = ((_sl1(jnp.tanh(dx1 - jnp.minimum(df[0], df[6])) * _S)
                + _sl1(jnp.tanh(dy1 - jnp.minimum(df[1], df[3])) * _S))
               + (_sl1(jnp.tanh(dx2 - jnp.maximum(df[2], df[4])) * _S)
                  + _sl1(jnp.tanh(dy2 - jnp.maximum(df[5], df[7])) * _S)))
    lb = jnp.sum(jnp.where(pos, b_terms, 0.0))

    cf = [conf_ref[s, c] for c in range(_C)]
    mrow = cf[0]
    for c in range(1, _C):
        mrow = jnp.maximum(mrow, cf[c])
    lse = jnp.log(_tree_add([jnp.exp(cf[c] - mrow) for c in range(_C)])) + mrow
    xt = _bitsel(cf, conf_t)
    ce = lse - xt
    ce_pos = jnp.sum(jnp.where(pos, ce, 0.0))
    npos = jnp.sum(pos.astype(jnp.int32)).astype(jnp.float32)

    lcm = jnp.where(pos | jnp.logical_not(valid), 0.0, ce)
    return ll, ce_pos, lfc, lb, npos, jnp.maximum(lcm, 0.0)


def _sample_body(tgt_ref, loc_ref, conf_ref, fc_ref, pri_ref, row_ref, lcm_ref,
                 *, num_priors, nobj):
    R, L = pri_ref.shape[1], pri_ref.shape[2]
    pcx, pcy, pw, ph = pri_ref[0], pri_ref[1], pri_ref[2], pri_ref[3]
    px1, py1, px2, py2 = pri_ref[4], pri_ref[5], pri_ref[6], pri_ref[7]
    pri = (pcx, pcy, pw, ph, px1, py1, px2, py2,
           (px2 - px1) * (py2 - py1))

    row_i = jax.lax.broadcasted_iota(jnp.int32, (R, L), 0)
    col_i = jax.lax.broadcasted_iota(jnp.int32, (R, L), 1)
    idx2 = row_i * L + col_i
    valid = idx2 < num_priors

    r8 = jax.lax.broadcasted_iota(jnp.int32, (8, 128), 0)
    c8 = jax.lax.broadcasted_iota(jnp.int32, (8, 128), 1)
    z = jnp.zeros((8, 128), jnp.float32)
    first = c8 == 0

    for s in range(_SPB):
        ll, ce_pos, lfc, lb, npos, lcm = _one_sample(
            s, tgt_ref, loc_ref, conf_ref, fc_ref, pri, idx2, valid,
            num_priors, nobj)
        lcm_ref[s] = lcm
        row_ref[s] = (jnp.where((r8 == 0) & first, ll, z)
                      + jnp.where((r8 == 1) & first, ce_pos, z)
                      + jnp.where((r8 == 2) & first, lfc, z)
                      + jnp.where((r8 == 3) & first, lb, z)
                      + jnp.where((r8 == 4) & first, npos, z))


def _topk_body(row_ref, lcm_ref, out_ref, *, num_priors):
    sb = row_ref[...]
    lcm = lcm_ref[...]
    B = sb.shape[0]
    npos = sb[:, 4, 0].astype(jnp.int32)
    kneg = jnp.minimum(3 * npos, num_priors - 1).reshape(B, 1, 1)

    u = pltpu.bitcast(lcm, jnp.int32)

    def _rowsum(x):
        return jnp.sum(jnp.sum(x, axis=1, keepdims=True), axis=2,
                       keepdims=True)

    def _bs(_, lohi):
        lo, hi = lohi
        mid = lo + ((hi - lo + 1) >> 1)
        cnt = _rowsum((u >= mid).astype(jnp.int32))
        ok = cnt >= kneg
        return (jnp.where(ok, mid, lo), jnp.where(ok, hi, mid - 1))

    init = (jnp.zeros((B, 1, 1), jnp.int32),
            jnp.full((B, 1, 1), 0x7F7FFFFF, jnp.int32))
    t, _ = jax.lax.fori_loop(0, 31, _bs, init)
    gt = u > t
    cnt_gt = _rowsum(gt.astype(jnp.int32))
    sum_gt = _rowsum(jnp.where(gt, lcm, 0.0))
    tval = jnp.max(jnp.max(jnp.where(u == t, lcm, 0.0), axis=1,
                           keepdims=True), axis=2, keepdims=True)
    topk = sum_gt + (kneg - cnt_gt).astype(jnp.float32) * tval

    ll = jnp.sum(sb[:, 0, 0])
    lc = jnp.sum(sb[:, 1, 0]) + jnp.sum(topk)
    lfc = jnp.sum(sb[:, 2, 0])
    lb = jnp.sum(sb[:, 3, 0])
    n = jnp.sum(sb[:, 4, 0])

    r8 = jax.lax.broadcasted_iota(jnp.int32, (8, 128), 0)
    c8 = jax.lax.broadcasted_iota(jnp.int32, (8, 128), 1)
    z = jnp.zeros((8, 128), jnp.float32)
    first = c8 == 0
    out_ref[...] = (jnp.where((r8 == 0) & first, ll, z)
                    + jnp.where((r8 == 1) & first, lc, z)
                    + jnp.where((r8 == 2) & first, lfc, z)
                    + jnp.where((r8 == 3) & first, lb, z)
                    + jnp.where((r8 == 4) & first, n, z))


def kernel(loc_data, conf_data, priors, four_corners_data, targets):
    B, P, C = conf_data.shape
    nobj = targets.shape[1]
    R = (-(-P // _L) + 7) // 8 * 8
    ppad = R * _L - P

    locp = jnp.pad(loc_data, ((0, 0), (0, ppad), (0, 0))) \
        .transpose(0, 2, 1).reshape(B, 4, R, _L)
    confp = jnp.pad(conf_data, ((0, 0), (0, ppad), (0, 0))) \
        .transpose(0, 2, 1).reshape(B, C, R, _L)
    fcp = jnp.pad(four_corners_data, ((0, 0), (0, ppad), (0, 0))) \
        .transpose(0, 2, 1).reshape(B, 8, R, _L)
    pf = jnp.concatenate(
        (priors[:, :2] - priors[:, 2:] / 2.0,
         priors[:, :2] + priors[:, 2:] / 2.0), axis=1)
    pri8 = jnp.pad(jnp.concatenate([priors, pf], axis=1).T,
                   ((0, 0), (0, ppad))).reshape(8, R, _L)

    rows, lcm = pl.pallas_call(
        functools.partial(_sample_body, num_priors=P, nobj=nobj),
        grid=(B // _SPB,),
        in_specs=[
            pl.BlockSpec((_SPB, nobj, 13), lambda b: (b, 0, 0),
                         memory_space=pltpu.SMEM),
            pl.BlockSpec((_SPB, 4, R, _L), lambda b: (b, 0, 0, 0)),
            pl.BlockSpec((_SPB, C, R, _L), lambda b: (b, 0, 0, 0)),
            pl.BlockSpec((_SPB, 8, R, _L), lambda b: (b, 0, 0, 0)),
            pl.BlockSpec((8, R, _L), lambda b: (0, 0, 0)),
        ],
        out_specs=[
            pl.BlockSpec((_SPB, 8, 128), lambda b: (b, 0, 0)),
            pl.BlockSpec((_SPB, R, _L), lambda b: (b, 0, 0)),
        ],
        out_shape=[
            jax.ShapeDtypeStruct((B, 8, 128), jnp.float32),
            jax.ShapeDtypeStruct((B, R, _L), jnp.float32),
        ],
        compiler_params=pltpu.CompilerParams(
            dimension_semantics=("parallel",)),
    )(targets, locp, confp, fcp, pri8)

    out = pl.pallas_call(
        functools.partial(_topk_body, num_priors=P),
        out_shape=jax.ShapeDtypeStruct((8, 128), jnp.float32),
    )(rows, lcm)

    n = out[4, 0]
    return (out[0, 0] / n, out[1, 0] / n, out[2, 0] / n, out[3, 0] / n)
